# Initial kernel scaffold; baseline (speedup 1.0000x reference)
import jax
import jax.numpy as jnp
from jax.experimental import pallas as pl


def kernel(x, edge_index, W1, b1, Wmu, bmu, Wlv, blv, W_lr, b_lr, W_lin, b_lin):
    raise NotImplementedError("write your pallas kernel here")


# speedup vs baseline: 41.6990x; 41.6990x over previous
import functools

import jax
import jax.numpy as jnp
from jax import lax
from jax.experimental import pallas as pl
from jax.experimental.pallas import tpu as pltpu
from jax.experimental.pallas import tpu_sc as plsc

N = 10000
E = 320000
T = 5
IN_FEAT = 128
HID = 16
LAT = 4
FB = T * 2 * LAT
OUT_FEAT = 64

NC = 2
NS = 16
NW = NC * NS
BATCH = 128
NB = 80
EP = NW * NB * BATCH
NPAD = 10240
ROWS_PER_TILE = NPAD // NS
NBK = 256
GN = NPAD // NBK


@functools.lru_cache(maxsize=None)
def _make_deg_kernel():
    mesh = plsc.VectorSubcoreMesh(core_axis_name="c", subcore_axis_name="s",
                                  num_cores=NC, num_subcores=NS)

    @functools.partial(
        pl.kernel,
        out_type=jax.ShapeDtypeStruct((NC, NPAD), jnp.float32),
        mesh=mesh,
        scratch_types=[
            pltpu.VMEM((NB, BATCH), jnp.int32),
            pltpu.VMEM((BATCH,), jnp.float32),
            pltpu.VMEM_SHARED((NPAD,), jnp.float32),
        ],
        compiler_params=pltpu.CompilerParams(use_tc_tiling_on_sc=False),
    )
    def deg_kernel(dstb, zeros1, out, dst_v, ones_v, acc):
        c = lax.axis_index("c")
        s = lax.axis_index("s")
        w = c * NS + s
        pltpu.sync_copy(dstb.at[w], dst_v)
        for k in range(BATCH // 16):
            ones_v[pl.ds(k * 16, 16)] = jnp.ones((16,), jnp.float32)
        pltpu.sync_copy(zeros1.at[pl.ds(s * ROWS_PER_TILE, ROWS_PER_TILE)],
                        acc.at[pl.ds(s * ROWS_PER_TILE, ROWS_PER_TILE)])
        plsc.subcore_barrier()

        def body(j, _):
            pltpu.sync_copy(ones_v, acc.at[dst_v.at[j]], add=True)
            return ()

        lax.fori_loop(0, NB, body, ())
        plsc.subcore_barrier()
        pltpu.sync_copy(acc.at[pl.ds(s * ROWS_PER_TILE, ROWS_PER_TILE)],
                        out.at[c, pl.ds(s * ROWS_PER_TILE, ROWS_PER_TILE)])

    return deg_kernel


@functools.lru_cache(maxsize=None)
def _make_prop_kernel(F, nbuf):
    mesh = plsc.VectorSubcoreMesh(core_axis_name="c", subcore_axis_name="s",
                                  num_cores=NC, num_subcores=NS)
    ngrp = NB // nbuf

    @functools.partial(
        pl.kernel,
        out_type=jax.ShapeDtypeStruct((NC, NPAD, F), jnp.float32),
        mesh=mesh,
        scratch_types=(
            [pltpu.VMEM((NB, BATCH), jnp.int32)] * 2
            + [pltpu.VMEM((BATCH, F), jnp.float32)] * nbuf
            + [pltpu.VMEM_SHARED((NPAD, F), jnp.float32)]
            + [pltpu.SemaphoreType.DMA] * (2 * nbuf)
        ),
        compiler_params=pltpu.CompilerParams(use_tc_tiling_on_sc=False),
    )
    def prop_kernel(table, srcb, dstb, zeros, out, src_v, dst_v, *rest):
        rbs = rest[:nbuf]
        acc = rest[nbuf]
        gsem = rest[nbuf + 1:2 * nbuf + 1]
        ssem = rest[2 * nbuf + 1:3 * nbuf + 1]
        c = lax.axis_index("c")
        s = lax.axis_index("s")
        w = c * NS + s
        pltpu.sync_copy(srcb.at[w], src_v)
        pltpu.sync_copy(dstb.at[w], dst_v)
        pltpu.sync_copy(zeros.at[pl.ds(s * ROWS_PER_TILE, ROWS_PER_TILE)],
                        acc.at[pl.ds(s * ROWS_PER_TILE, ROWS_PER_TILE)])
        plsc.subcore_barrier()

        for b in range(nbuf):
            pltpu.async_copy(table.at[src_v.at[b]], rbs[b], gsem[b])

        def _fire_next_gather(i, b):
            @pl.when(i + 1 < ngrp)
            def _():
                j = nbuf * (i + 1) + b
                pltpu.async_copy(table.at[src_v.at[j]], rbs[b], gsem[b])

        def body(i, _):
            base = nbuf * i
            for b in range(nbuf):
                j = base + b
                pltpu.make_async_copy(table.at[src_v.at[j]], rbs[b],
                                      gsem[b]).wait()
                pltpu.async_copy(rbs[b], acc.at[dst_v.at[j]], ssem[b],
                                 add=True)
            for b in range(nbuf):
                j = base + b
                pltpu.make_async_copy(rbs[b], acc.at[dst_v.at[j]],
                                      ssem[b]).wait()
                _fire_next_gather(i, b)
            return ()

        lax.fori_loop(0, ngrp, body, ())
        plsc.subcore_barrier()
        pltpu.sync_copy(acc.at[pl.ds(s * ROWS_PER_TILE, ROWS_PER_TILE)],
                        out.at[c, pl.ds(s * ROWS_PER_TILE, ROWS_PER_TILE)])

    return prop_kernel


def _tc1_body(x_ref, deg_ref, w1_ref, wc_ref, v_ref, dinv_ref):
    deg = deg_ref[0] + deg_ref[1] + 1.0
    dinv = lax.rsqrt(jnp.maximum(deg, 1.0))
    for t in range(T):
        g = jnp.dot(x_ref[t], w1_ref[...], preferred_element_type=jnp.float32)
        m = jnp.dot(g, wc_ref[...], preferred_element_type=jnp.float32)
        v_ref[:, t * 2 * LAT:(t + 1) * 2 * LAT] = m * dinv
    dinv_ref[...] = dinv


def _tc1(x, degS, W1, Wc):
    return pl.pallas_call(
        _tc1_body,
        grid=(GN,),
        in_specs=[
            pl.BlockSpec((T, NBK, IN_FEAT), lambda nb: (0, nb, 0)),
            pl.BlockSpec((NC, NBK, 1), lambda nb: (0, nb, 0)),
            pl.BlockSpec((IN_FEAT, HID), lambda nb: (0, 0)),
            pl.BlockSpec((HID, 2 * LAT), lambda nb: (0, 0)),
        ],
        out_specs=[
            pl.BlockSpec((NBK, FB), lambda nb: (nb, 0)),
            pl.BlockSpec((NBK, 1), lambda nb: (nb, 0)),
        ],
        out_shape=[
            jax.ShapeDtypeStruct((N, FB), jnp.float32),
            jax.ShapeDtypeStruct((NPAD, 1), jnp.float32),
        ],
    )(x, degS, W1, Wc)


def _tc2_body(s_ref, v_ref, dinv_ref, b1_ref, wc_ref, v2_ref):
    dinv = dinv_ref[...]
    agg = (s_ref[0] + s_ref[1] + v_ref[...]) * dinv
    c = jnp.dot(b1_ref[...], wc_ref[...],
                preferred_element_type=jnp.float32)
    ct = jnp.concatenate([c] * T, axis=1)
    v2_ref[...] = (agg + ct) * dinv


def _tc2(S, V, dinv, b1, Wc):
    return pl.pallas_call(
        _tc2_body,
        grid=(GN,),
        in_specs=[
            pl.BlockSpec((NC, NBK, FB), lambda nb: (0, nb, 0)),
            pl.BlockSpec((NBK, FB), lambda nb: (nb, 0)),
            pl.BlockSpec((NBK, 1), lambda nb: (nb, 0)),
            pl.BlockSpec((1, HID), lambda nb: (0, 0)),
            pl.BlockSpec((HID, 2 * LAT), lambda nb: (0, 0)),
        ],
        out_specs=pl.BlockSpec((NBK, FB), lambda nb: (nb, 0)),
        out_shape=jax.ShapeDtypeStruct((N, FB), jnp.float32),
    )(S, V, dinv, b1, Wc)


def _tc3a_body(s_ref, mp_ref, dinv_ref, eps_ref, bmu_ref, blv_ref,
               z_ref, mum_ref, lvm_ref):
    p = (s_ref[0] + s_ref[1] + mp_ref[...]) * dinv_ref[...]
    mus = 0.0
    lvs = 0.0
    for t in range(T):
        mu = p[:, t * 2 * LAT:t * 2 * LAT + LAT] + bmu_ref[...]
        lv = p[:, t * 2 * LAT + LAT:(t + 1) * 2 * LAT] + blv_ref[...]
        z_ref[t] = mu + eps_ref[t] * jnp.exp(0.5 * lv)
        mus = mus + mu
        lvs = lvs + lv
    mum_ref[...] = mus * (1.0 / T)
    lvm_ref[...] = lvs * (1.0 / T)


def _tc3a(S, Mp, dinv, eps, bmu, blv):
    return pl.pallas_call(
        _tc3a_body,
        grid=(GN,),
        in_specs=[
            pl.BlockSpec((NC, NBK, FB), lambda nb: (0, nb, 0)),
            pl.BlockSpec((NBK, FB), lambda nb: (nb, 0)),
            pl.BlockSpec((NBK, 1), lambda nb: (nb, 0)),
            pl.BlockSpec((T, NBK, LAT), lambda nb: (0, nb, 0)),
            pl.BlockSpec((1, LAT), lambda nb: (0, 0)),
            pl.BlockSpec((1, LAT), lambda nb: (0, 0)),
        ],
        out_specs=[
            pl.BlockSpec((T, NBK, LAT), lambda nb: (0, nb, 0)),
            pl.BlockSpec((NBK, LAT), lambda nb: (nb, 0)),
            pl.BlockSpec((NBK, LAT), lambda nb: (nb, 0)),
        ],
        out_shape=[
            jax.ShapeDtypeStruct((T, N, LAT), jnp.float32),
            jax.ShapeDtypeStruct((N, LAT), jnp.float32),
            jax.ShapeDtypeStruct((N, LAT), jnp.float32),
        ],
    )(S, Mp, dinv, eps, bmu, blv)


KB = 4000
GK = (T * N * LAT) // KB


def _round_bf16(v):
    u = lax.bitcast_convert_type(v, jnp.uint32)
    r = (u + jnp.uint32(0x7FFF) + ((u >> 16) & jnp.uint32(1))) \
        & jnp.uint32(0xFFFF0000)
    return lax.bitcast_convert_type(r, jnp.float32)


def _tc3b_body(z_ref, w_ref, blr_ref, wlin_ref, blin_ref, out_ref, acc_ref):
    k = pl.program_id(0)
    zb = _round_bf16(z_ref[...])
    wb = _round_bf16(w_ref[...])
    part = jnp.sum(zb * wb, axis=0, keepdims=True)

    @pl.when(k == 0)
    def _():
        acc_ref[...] = part

    @pl.when(k > 0)
    def _():
        acc_ref[...] += part

    @pl.when(k == GK - 1)
    def _():
        s = jax.nn.sigmoid(acc_ref[...] + blr_ref[...])
        out_ref[...] = (jnp.dot(s, wlin_ref[...],
                                preferred_element_type=jnp.float32)
                        + blin_ref[...])


def _tc3b(z2, W_lr, b_lr, W_lin, b_lin):
    return pl.pallas_call(
        _tc3b_body,
        grid=(GK,),
        in_specs=[
            pl.BlockSpec((KB, 1), lambda k: (k, 0)),
            pl.BlockSpec((KB, OUT_FEAT), lambda k: (k, 0)),
            pl.BlockSpec((1, OUT_FEAT), lambda k: (0, 0)),
            pl.BlockSpec((OUT_FEAT, 1), lambda k: (0, 0)),
            pl.BlockSpec((1, 1), lambda k: (0, 0)),
        ],
        out_specs=pl.BlockSpec((1, 1), lambda k: (0, 0)),
        out_shape=jax.ShapeDtypeStruct((1, 1), jnp.float32),
        scratch_shapes=[pltpu.VMEM((1, OUT_FEAT), jnp.float32)],
    )(z2, W_lr, b_lr, W_lin, b_lin)


def kernel(x, edge_index, W1, b1, Wmu, bmu, Wlv, blv, W_lr, b_lr, W_lin, b_lin):
    src = edge_index[0]
    dst = edge_index[1]
    pad = EP - E
    pad_dst = N + (jnp.arange(pad, dtype=jnp.int32) % (NPAD - N))
    srcb = jnp.concatenate([src, jnp.zeros((pad,), jnp.int32)]).reshape(
        NW, NB, BATCH)
    dstb = jnp.concatenate([dst, pad_dst]).reshape(NW, NB, BATCH)

    zeros1 = jnp.zeros((NPAD,), jnp.float32)
    zerosF = jnp.zeros((NPAD, FB), jnp.float32)

    eps = jnp.stack([
        jax.random.normal(jax.random.fold_in(jax.random.key(42), t),
                          (N, LAT), dtype=jnp.float32)
        for t in range(T)
    ])

    Wc = jnp.concatenate([Wmu, Wlv], axis=1)
    degS = _make_deg_kernel()(dstb, zeros1)
    V, dinv = _tc1(x, degS.reshape(NC, NPAD, 1), W1, Wc)
    SA = _make_prop_kernel(FB, 8)(V, srcb, dstb, zerosF)
    V2 = _tc2(SA, V, dinv, b1.reshape(1, HID), Wc)
    SB = _make_prop_kernel(FB, 8)(V2, srcb, dstb, zerosF)
    z, mu_mean, lv_mean = _tc3a(SB, V2, dinv, eps,
                                bmu.reshape(1, LAT), blv.reshape(1, LAT))
    z2 = z.reshape(T * N * LAT, 1)
    out = _tc3b(z2, W_lr, b_lr.reshape(1, OUT_FEAT), W_lin,
                b_lin.reshape(1, 1))
    return out, mu_mean, lv_mean
